# baseline (speedup 1.0000x reference)
import jax
import jax.numpy as jnp
from jax.experimental import pallas as pl
from jax.experimental.pallas import tpu as pltpu

IN_F = 512
OUT_F = 512
TM = 512


def _gcn_kernel(x_ref, adj_ref, w_ref, b_ref, out_ref, xbf_ref):
    m = pl.program_id(1)

    @pl.when(m == 0)
    def _():
        xbf_ref[...] = x_ref[0].astype(jnp.bfloat16)

    a = adj_ref[0].astype(jnp.bfloat16)
    t = jnp.dot(a, xbf_ref[...], preferred_element_type=jnp.float32)
    acc = jnp.dot(t.astype(jnp.bfloat16), w_ref[...].astype(jnp.bfloat16),
                  preferred_element_type=jnp.float32)
    out_ref[0] = acc + b_ref[...]


def kernel(input, adj, W, b):
    B, N, _ = input.shape
    grid = (B, N // TM)
    b2d = b.reshape(1, OUT_F)
    return pl.pallas_call(
        _gcn_kernel,
        grid=grid,
        in_specs=[
            pl.BlockSpec((1, N, IN_F), lambda i, m: (i, 0, 0)),
            pl.BlockSpec((1, TM, N), lambda i, m: (i, m, 0)),
            pl.BlockSpec((IN_F, OUT_F), lambda i, m: (0, 0)),
            pl.BlockSpec((1, OUT_F), lambda i, m: (0, 0)),
        ],
        out_specs=pl.BlockSpec((1, TM, OUT_F), lambda i, m: (i, m, 0)),
        out_shape=jax.ShapeDtypeStruct((B, N, OUT_F), jnp.float32),
        scratch_shapes=[pltpu.VMEM((N, IN_F), jnp.bfloat16)],
        compiler_params=pltpu.CompilerParams(
            dimension_semantics=("arbitrary", "arbitrary"),
        ),
    )(input, adj, W, b2d)


# speedup vs baseline: 1.0019x; 1.0019x over previous
import jax
import jax.numpy as jnp
from jax.experimental import pallas as pl
from jax.experimental.pallas import tpu as pltpu

IN_F = 512
OUT_F = 512
TM = 512


def _gcn_kernel(x_ref, adj_ref, w_ref, b_ref, out_ref, xbf_ref):
    m = pl.program_id(1)

    @pl.when(m == 0)
    def _():
        xbf_ref[...] = x_ref[0].astype(jnp.bfloat16)

    t = jax.lax.dot_general(
        adj_ref[0], xbf_ref[...],
        (((1,), (0,)), ((), ())),
        precision=jax.lax.Precision.DEFAULT,
        preferred_element_type=jnp.float32)
    acc = jax.lax.dot_general(
        t, w_ref[...],
        (((1,), (0,)), ((), ())),
        precision=jax.lax.Precision.DEFAULT,
        preferred_element_type=jnp.float32)
    out_ref[0] = acc + b_ref[...]


def kernel(input, adj, W, b):
    B, N, _ = input.shape
    grid = (B, N // TM)
    b2d = b.reshape(1, OUT_F)
    return pl.pallas_call(
        _gcn_kernel,
        grid=grid,
        in_specs=[
            pl.BlockSpec((1, N, IN_F), lambda i, m: (i, 0, 0)),
            pl.BlockSpec((1, TM, N), lambda i, m: (i, m, 0)),
            pl.BlockSpec((IN_F, OUT_F), lambda i, m: (0, 0)),
            pl.BlockSpec((1, OUT_F), lambda i, m: (0, 0)),
        ],
        out_specs=pl.BlockSpec((1, TM, OUT_F), lambda i, m: (i, m, 0)),
        out_shape=jax.ShapeDtypeStruct((B, N, OUT_F), jnp.float32),
        scratch_shapes=[pltpu.VMEM((N, IN_F), jnp.bfloat16)],
        compiler_params=pltpu.CompilerParams(
            dimension_semantics=("arbitrary", "arbitrary"),
        ),
    )(input, adj, W, b2d)


# speedup vs baseline: 1.0144x; 1.0125x over previous
import jax
import jax.numpy as jnp
from jax.experimental import pallas as pl
from jax.experimental.pallas import tpu as pltpu

IN_F = 512
OUT_F = 512
TM = 512


def _gcn_kernel(x_ref, adj_ref, w_ref, b_ref, out_ref):
    t = jax.lax.dot_general(
        adj_ref[0], x_ref[0],
        (((1,), (0,)), ((), ())),
        precision=jax.lax.Precision.DEFAULT,
        preferred_element_type=jnp.float32)
    acc = jax.lax.dot_general(
        t, w_ref[...],
        (((1,), (0,)), ((), ())),
        precision=jax.lax.Precision.DEFAULT,
        preferred_element_type=jnp.float32)
    out_ref[0] = acc + b_ref[...]


def kernel(input, adj, W, b):
    B, N, _ = input.shape
    grid = (B, N // TM)
    b2d = b.reshape(1, OUT_F)
    return pl.pallas_call(
        _gcn_kernel,
        grid=grid,
        in_specs=[
            pl.BlockSpec((1, N, IN_F), lambda i, m: (i, 0, 0)),
            pl.BlockSpec((1, TM, N), lambda i, m: (i, m, 0)),
            pl.BlockSpec((IN_F, OUT_F), lambda i, m: (0, 0)),
            pl.BlockSpec((1, OUT_F), lambda i, m: (0, 0)),
        ],
        out_specs=pl.BlockSpec((1, TM, OUT_F), lambda i, m: (i, m, 0)),
        out_shape=jax.ShapeDtypeStruct((B, N, OUT_F), jnp.float32),
        compiler_params=pltpu.CompilerParams(
            dimension_semantics=("arbitrary", "arbitrary"),
        ),
    )(input, adj, W, b2d)
